# baseline (speedup 1.0000x reference)
import functools

import jax
import jax.numpy as jnp
from jax import lax
from jax.experimental import pallas as pl
from jax.experimental.pallas import tpu as pltpu
from jax.experimental.pallas import tpu_sc as plsc

_N = 10000
_E = 320000
_D = 128
_NW = 32
_EPW = _E // _NW
_C = 80
_NCH = _EPW // _C
_G = _C // 16


@functools.partial(
    pl.kernel,
    mesh=plsc.VectorSubcoreMesh(core_axis_name="c", subcore_axis_name="s"),
    out_type=jax.ShapeDtypeStruct((_E,), jnp.float32),
    compiler_params=pltpu.CompilerParams(needs_layout_passes=False),
    scratch_types=[
        pltpu.VMEM_SHARED((_N, _D), jnp.float32),
        [pltpu.VMEM((_C,), jnp.int32) for _ in range(4)],
        [pltpu.VMEM((_C,), jnp.int32) for _ in range(4)],
        [pltpu.VMEM((_C, _D), jnp.float32) for _ in range(2)],
        [pltpu.VMEM((_C, _D), jnp.float32) for _ in range(2)],
        [pltpu.VMEM((_C,), jnp.float32) for _ in range(2)],
        pltpu.VMEM((256,), jnp.float32),
        [pltpu.SemaphoreType.DMA for _ in range(4)],
        [pltpu.SemaphoreType.DMA for _ in range(2)],
        [pltpu.SemaphoreType.DMA for _ in range(2)],
    ],
)
def _edge_dot(d0_hbm, d1_hbm, table_hbm, out_hbm,
              table_sh, idx0_bufs, idx1_bufs, rows0_bufs, rows1_bufs,
              out_bufs, tbuf_v, isems, gsems, osems):
    cid = lax.axis_index("c")
    sid = lax.axis_index("s")
    wid = sid * 2 + cid
    base_w = wid * _EPW

    rows_per_sub = 624
    pltpu.sync_copy(table_hbm.at[pl.ds(sid * rows_per_sub, rows_per_sub)],
                    table_sh.at[pl.ds(sid * rows_per_sub, rows_per_sub)])

    @pl.when(sid == 0)
    def _():
        rem = 16 * rows_per_sub
        pltpu.sync_copy(table_hbm.at[pl.ds(rem, _N - rem)],
                        table_sh.at[pl.ds(rem, _N - rem)])

    plsc.subcore_barrier()

    def issue_idx(t, b4):
        off = base_w + t * _C
        pltpu.async_copy(d0_hbm.at[pl.ds(off, _C)], idx0_bufs[b4], isems[b4])
        pltpu.async_copy(d1_hbm.at[pl.ds(off, _C)], idx1_bufs[b4], isems[b4])

    def wait_idx(b4):
        pltpu.make_async_copy(d0_hbm.at[pl.ds(0, _C)], idx0_bufs[b4],
                              isems[b4]).wait()
        pltpu.make_async_copy(d1_hbm.at[pl.ds(0, _C)], idx1_bufs[b4],
                              isems[b4]).wait()

    def issue_gather(b4, b2):
        pltpu.async_copy(table_sh.at[idx0_bufs[b4]], rows0_bufs[b2], gsems[b2])
        pltpu.async_copy(table_sh.at[idx1_bufs[b4]], rows1_bufs[b2], gsems[b2])

    def drain_gather(b2):
        pltpu.make_async_copy(table_hbm.at[pl.ds(0, _C)],
                              rows0_bufs[b2], gsems[b2]).wait()
        pltpu.make_async_copy(table_hbm.at[pl.ds(0, _C)],
                              rows1_bufs[b2], gsems[b2]).wait()

    def drain_out(b2):
        pltpu.make_async_copy(out_bufs[b2], out_hbm.at[pl.ds(0, _C)],
                              osems[b2]).wait()

    def compute(b2):
        r0, r1 = rows0_bufs[b2], rows1_bufs[b2]
        ob = out_bufs[b2]

        def group_body(g, carry):
            for e16 in range(16):
                e = g * 16 + e16
                acc = jnp.zeros((16,), jnp.float32)
                for k in range(_D // 16):
                    acc = acc + r0[e, pl.ds(k * 16, 16)] * r1[e, pl.ds(k * 16, 16)]
                tbuf_v[pl.ds(e16 * 16, 16)] = acc
            ids = lax.iota(jnp.int32, 16) * 16
            o = jnp.zeros((16,), jnp.float32)
            for l in range(16):
                o = o + plsc.load_gather(tbuf_v, [ids + l])
            ob[pl.ds(g * 16, 16)] = o
            return carry

        lax.fori_loop(0, _G, group_body, 0)

    def issue_out(t, b2):
        pltpu.async_copy(out_bufs[b2],
                         out_hbm.at[pl.ds(base_w + t * _C, _C)], osems[b2])

    for c in range(4):
        issue_idx(c, c)
    for b in range(2):
        wait_idx(b)
        issue_gather(b, b)

    def loop_body(tt, carry):
        for b in range(4):
            t = tt * 4 + b
            b2 = b % 2
            drain_gather(b2)

            @pl.when(t + 4 < _NCH)
            def _():
                issue_idx(t + 4, b)

            @pl.when(t >= 2)
            def _():
                drain_out(b2)

            compute(b2)
            issue_out(t, b2)

            @pl.when(t + 2 < _NCH)
            def _():
                wait_idx((b + 2) % 4)
                issue_gather((b + 2) % 4, b2)

        return carry

    lax.fori_loop(0, _NCH // 4, loop_body, 0)

    drain_gather(0)
    drain_out(0)
    compute(0)
    issue_out(_NCH - 1, 0)

    drain_out(0)
    drain_out(1)


def kernel(data, embedding):
    return _edge_dot(data[0], data[1], embedding)


# speedup vs baseline: 1.4437x; 1.4437x over previous
import functools

import jax
import jax.numpy as jnp
from jax import lax
from jax.experimental import pallas as pl
from jax.experimental.pallas import tpu as pltpu
from jax.experimental.pallas import tpu_sc as plsc

_N = 10000
_E = 320000
_D = 128
_NW = 32
_EPW = _E // _NW
_C = 80
_NCH = _EPW // _C
_G = _C // 16


@functools.partial(
    pl.kernel,
    mesh=plsc.VectorSubcoreMesh(core_axis_name="c", subcore_axis_name="s"),
    out_type=jax.ShapeDtypeStruct((_E,), jnp.float32),
    compiler_params=pltpu.CompilerParams(needs_layout_passes=False),
    scratch_types=[
        pltpu.VMEM_SHARED((_N, _D), jnp.float32),
        [pltpu.VMEM((_C,), jnp.int32) for _ in range(4)],
        [pltpu.VMEM((_C,), jnp.int32) for _ in range(4)],
        [pltpu.VMEM((_C, _D), jnp.float32) for _ in range(2)],
        [pltpu.VMEM((_C, _D), jnp.float32) for _ in range(2)],
        [pltpu.VMEM((_C,), jnp.float32) for _ in range(2)],
        pltpu.VMEM((_C * 16,), jnp.float32),
        [pltpu.SemaphoreType.DMA for _ in range(4)],
        [pltpu.SemaphoreType.DMA for _ in range(2)],
        [pltpu.SemaphoreType.DMA for _ in range(2)],
    ],
)
def _edge_dot(d0_hbm, d1_hbm, table_hbm, out_hbm,
              table_sh, idx0_bufs, idx1_bufs, rows0_bufs, rows1_bufs,
              out_bufs, tbuf_v, isems, gsems, osems):
    cid = lax.axis_index("c")
    sid = lax.axis_index("s")
    wid = sid * 2 + cid
    base_w = wid * _EPW

    rows_per_sub = 624
    pltpu.sync_copy(table_hbm.at[pl.ds(sid * rows_per_sub, rows_per_sub)],
                    table_sh.at[pl.ds(sid * rows_per_sub, rows_per_sub)])

    @pl.when(sid == 0)
    def _():
        rem = 16 * rows_per_sub
        pltpu.sync_copy(table_hbm.at[pl.ds(rem, _N - rem)],
                        table_sh.at[pl.ds(rem, _N - rem)])

    plsc.subcore_barrier()

    def issue_idx(t, b4):
        off = base_w + t * _C
        pltpu.async_copy(d0_hbm.at[pl.ds(off, _C)], idx0_bufs[b4], isems[b4])
        pltpu.async_copy(d1_hbm.at[pl.ds(off, _C)], idx1_bufs[b4], isems[b4])

    def wait_idx(b4):
        pltpu.make_async_copy(d0_hbm.at[pl.ds(0, _C)], idx0_bufs[b4],
                              isems[b4]).wait()
        pltpu.make_async_copy(d1_hbm.at[pl.ds(0, _C)], idx1_bufs[b4],
                              isems[b4]).wait()

    def issue_gather(b4, b2):
        pltpu.async_copy(table_sh.at[idx0_bufs[b4]], rows0_bufs[b2], gsems[b2])
        pltpu.async_copy(table_sh.at[idx1_bufs[b4]], rows1_bufs[b2], gsems[b2])

    def drain_gather(b2):
        pltpu.make_async_copy(table_hbm.at[pl.ds(0, _C)],
                              rows0_bufs[b2], gsems[b2]).wait()
        pltpu.make_async_copy(table_hbm.at[pl.ds(0, _C)],
                              rows1_bufs[b2], gsems[b2]).wait()

    def drain_out(b2):
        pltpu.make_async_copy(out_bufs[b2], out_hbm.at[pl.ds(0, _C)],
                              osems[b2]).wait()

    def compute(b2):
        r0, r1 = rows0_bufs[b2], rows1_bufs[b2]
        ob = out_bufs[b2]

        @plsc.parallel_loop(0, _C, step=1, unroll=4)
        def edge_body(e):
            p = [r0[e, pl.ds(k * 16, 16)] * r1[e, pl.ds(k * 16, 16)]
                 for k in range(_D // 16)]
            while len(p) > 1:
                p = [p[i] + p[i + 1] for i in range(0, len(p), 2)]
            tbuf_v[pl.ds(e * 16, 16)] = p[0]

        @plsc.parallel_loop(0, _G, step=1, unroll=1)
        def group_body(g):
            ids = lax.iota(jnp.int32, 16) * 16 + g * 256
            o = jnp.zeros((16,), jnp.float32)
            for l in range(16):
                o = o + plsc.load_gather(tbuf_v, [ids + l])
            ob[pl.ds(g * 16, 16)] = o

    def issue_out(t, b2):
        pltpu.async_copy(out_bufs[b2],
                         out_hbm.at[pl.ds(base_w + t * _C, _C)], osems[b2])

    for c in range(4):
        issue_idx(c, c)
    for b in range(2):
        wait_idx(b)
        issue_gather(b, b)

    def loop_body(tt, carry):
        for b in range(4):
            t = tt * 4 + b
            b2 = b % 2
            drain_gather(b2)

            @pl.when(t + 4 < _NCH)
            def _():
                issue_idx(t + 4, b)

            @pl.when(t >= 2)
            def _():
                drain_out(b2)

            compute(b2)
            issue_out(t, b2)

            @pl.when(t + 2 < _NCH)
            def _():
                wait_idx((b + 2) % 4)
                issue_gather((b + 2) % 4, b2)

        return carry

    lax.fori_loop(0, _NCH // 4, loop_body, 0)

    drain_gather(0)
    drain_out(0)
    compute(0)
    issue_out(_NCH - 1, 0)

    drain_out(0)
    drain_out(1)


def kernel(data, embedding):
    return _edge_dot(data[0], data[1], embedding)
